# Initial kernel scaffold; baseline (speedup 1.0000x reference)
import jax
import jax.numpy as jnp
from jax.experimental import pallas as pl


def kernel(x, label, weight):
    raise NotImplementedError("write your pallas kernel here")


# speedup vs baseline: 1.3949x; 1.3949x over previous
import functools

import jax
import jax.numpy as jnp
import numpy as np
from jax import lax
from jax.experimental import pallas as pl
from jax.experimental.pallas import tpu as pltpu
from jax.experimental.pallas import tpu_sc as plsc

NC, NS, L = 2, 16, 16
NW = NC * NS
BATCH, D, V = 16384, 512, 1000
PW = D // 32
NCHUNK_F32 = D // L
B_PER_W = BATCH // NW
CHUNK = 64
ROWS_PER_W_PACK = 32

_mesh = plsc.VectorSubcoreMesh(core_axis_name="c", subcore_axis_name="s")
_params = pltpu.CompilerParams(needs_layout_passes=False)

_BIT = [1 << c for c in range(32)]


def _wid():
    return lax.axis_index("s") * NC + lax.axis_index("c")


@functools.partial(
    pl.kernel,
    out_type=jax.ShapeDtypeStruct((V * PW,), jnp.int32),
    mesh=_mesh,
    compiler_params=_params,
    scratch_types=[
        pltpu.VMEM((ROWS_PER_W_PACK, D), jnp.float32),
        pltpu.VMEM((ROWS_PER_W_PACK * PW,), jnp.int32),
    ],
)
def _pack_kernel(w_hbm, packed_hbm, w_v, packed_v):
    wid = _wid()
    start = jnp.minimum(wid * ROWS_PER_W_PACK, V - ROWS_PER_W_PACK)
    pltpu.sync_copy(w_hbm.at[pl.ds(start, ROWS_PER_W_PACK)], w_v)

    def row_body(r, _):
        bits = jnp.zeros((L,), jnp.uint32)
        for c in range(32):
            wv = w_v[r, pl.ds(c * L, L)]
            bits = bits | jnp.where(wv > 0.5, jnp.uint32(_BIT[c]),
                                    jnp.uint32(0))
        packed_v[pl.ds(r * PW, PW)] = plsc.bitcast(bits, jnp.int32)
        return 0

    lax.fori_loop(0, ROWS_PER_W_PACK, row_body, 0)
    pltpu.sync_copy(packed_v,
                    packed_hbm.at[pl.ds(start * PW, ROWS_PER_W_PACK * PW)])


@functools.partial(
    pl.kernel,
    out_type=jax.ShapeDtypeStruct((BATCH, D), jnp.float32),
    mesh=_mesh,
    compiler_params=_params,
    scratch_types=[
        pltpu.VMEM((V * PW,), jnp.int32),
        pltpu.VMEM((B_PER_W,), jnp.int32),
        pltpu.VMEM((CHUNK, D), jnp.float32),
        pltpu.VMEM((CHUNK, D), jnp.float32),
    ],
)
def _main_kernel(x_hbm, lbl_hbm, packed_hbm, out_hbm,
                 ptab_v, lbl_v, x_v, out_v):
    base_w = _wid() * B_PER_W
    pltpu.sync_copy(packed_hbm, ptab_v)
    pltpu.sync_copy(lbl_hbm.at[pl.ds(base_w, B_PER_W)], lbl_v)
    lane_ids = lax.iota(jnp.int32, L)

    def chunk_body(g, _):
        base = base_w + g * CHUNK
        pltpu.sync_copy(x_hbm.at[pl.ds(base, CHUNK)], x_v)

        def row_body(i, _):
            row_splat = plsc.load_gather(
                lbl_v, [jnp.full((L,), g * CHUNK + i, jnp.int32)])
            pv = plsc.bitcast(
                plsc.load_gather(ptab_v, [row_splat * PW + lane_ids]),
                jnp.uint32)
            for c in range(NCHUNK_F32):
                xv = x_v[i, pl.ds(c * L, L)]
                m = (pv & jnp.uint32(_BIT[c])) != 0
                out_v[i, pl.ds(c * L, L)] = jnp.where(m, xv, 0.0)
            return 0

        lax.fori_loop(0, CHUNK, row_body, 0)
        pltpu.sync_copy(out_v, out_hbm.at[pl.ds(base, CHUNK)])
        return 0

    lax.fori_loop(0, B_PER_W // CHUNK, chunk_body, 0)


def kernel(x, label, weight):
    packed = _pack_kernel(weight)
    return _main_kernel(x, label, packed)


# speedup vs baseline: 1.5606x; 1.1188x over previous
import functools

import jax
import jax.numpy as jnp
import numpy as np
from jax import lax
from jax.experimental import pallas as pl
from jax.experimental.pallas import tpu as pltpu
from jax.experimental.pallas import tpu_sc as plsc

NC, NS, L = 2, 16, 16
NW = NC * NS
BATCH, D, V = 16384, 512, 1000
PW = D // 32
NCHUNK_F32 = D // L
B_PER_W = BATCH // NW
CHUNK = 32
ROWS_PER_W_PACK = 32

_mesh = plsc.VectorSubcoreMesh(core_axis_name="c", subcore_axis_name="s")
_params = pltpu.CompilerParams(needs_layout_passes=False)

_BIT = [1 << c for c in range(32)]


def _wid():
    return lax.axis_index("s") * NC + lax.axis_index("c")


@functools.partial(
    pl.kernel,
    out_type=jax.ShapeDtypeStruct((V * PW,), jnp.int32),
    mesh=_mesh,
    compiler_params=_params,
    scratch_types=[
        pltpu.VMEM((ROWS_PER_W_PACK, D), jnp.float32),
        pltpu.VMEM((ROWS_PER_W_PACK * PW,), jnp.int32),
    ],
)
def _pack_kernel(w_hbm, packed_hbm, w_v, packed_v):
    wid = _wid()
    start = jnp.minimum(wid * ROWS_PER_W_PACK, V - ROWS_PER_W_PACK)
    pltpu.sync_copy(w_hbm.at[pl.ds(start, ROWS_PER_W_PACK)], w_v)

    def row_body(r, _):
        bits = jnp.zeros((L,), jnp.uint32)
        for c in range(32):
            wv = w_v[r, pl.ds(c * L, L)]
            bits = bits | jnp.where(wv > 0.5, jnp.uint32(_BIT[c]),
                                    jnp.uint32(0))
        packed_v[pl.ds(r * PW, PW)] = plsc.bitcast(bits, jnp.int32)
        return 0

    lax.fori_loop(0, ROWS_PER_W_PACK, row_body, 0)
    pltpu.sync_copy(packed_v,
                    packed_hbm.at[pl.ds(start * PW, ROWS_PER_W_PACK * PW)])


NG = B_PER_W // CHUNK


@functools.partial(
    pl.kernel,
    out_type=jax.ShapeDtypeStruct((BATCH, D), jnp.float32),
    mesh=_mesh,
    compiler_params=_params,
    scratch_types=[
        pltpu.VMEM((V * PW,), jnp.int32),
        pltpu.VMEM((B_PER_W,), jnp.int32),
        pltpu.VMEM((2, CHUNK, D), jnp.float32),
        pltpu.VMEM((2, CHUNK, D), jnp.float32),
        pltpu.SemaphoreType.DMA,
        pltpu.SemaphoreType.DMA,
        pltpu.SemaphoreType.DMA,
        pltpu.SemaphoreType.DMA,
    ],
)
def _main_kernel(x_hbm, lbl_hbm, packed_hbm, out_hbm,
                 ptab_v, lbl_v, x_v, out_v, sx0, sx1, so0, so1):
    base_w = _wid() * B_PER_W
    pltpu.sync_copy(packed_hbm, ptab_v)
    pltpu.sync_copy(lbl_hbm.at[pl.ds(base_w, B_PER_W)], lbl_v)
    lane_ids = lax.iota(jnp.int32, L)
    sx, so = (sx0, sx1), (so0, so1)

    def x_copy(g):
        return pltpu.make_async_copy(
            x_hbm.at[pl.ds(base_w + g * CHUNK, CHUNK)], x_v.at[g % 2],
            sx[g % 2])

    def out_copy(g):
        return pltpu.make_async_copy(
            out_v.at[g % 2], out_hbm.at[pl.ds(base_w + g * CHUNK, CHUNK)],
            so[g % 2])

    x_copy(0).start()
    for g in range(NG):
        b = g % 2
        if g + 1 < NG:
            x_copy(g + 1).start()
        x_copy(g).wait()
        if g >= 2:
            out_copy(g - 2).wait()

        def row_body(i, _):
            row_splat = plsc.load_gather(
                lbl_v, [jnp.full((L,), g * CHUNK + i, jnp.int32)])
            pv = plsc.bitcast(
                plsc.load_gather(ptab_v, [row_splat * PW + lane_ids]),
                jnp.uint32)
            for c in range(NCHUNK_F32):
                xv = x_v[b, i, pl.ds(c * L, L)]
                m = (pv & jnp.uint32(_BIT[c])) != 0
                out_v[b, i, pl.ds(c * L, L)] = jnp.where(m, xv, 0.0)
            return 0

        lax.fori_loop(0, CHUNK, row_body, 0)
        out_copy(g).start()
    out_copy(NG - 2).wait()
    out_copy(NG - 1).wait()


def kernel(x, label, weight):
    packed = _pack_kernel(weight)
    return _main_kernel(x, label, packed)


# speedup vs baseline: 1.6436x; 1.0532x over previous
import functools

import jax
import jax.numpy as jnp
from jax import lax
from jax.experimental import pallas as pl
from jax.experimental.pallas import tpu as pltpu
from jax.experimental.pallas import tpu_sc as plsc

NC, NS, L = 2, 16, 16
NW = NC * NS
BATCH, D, V = 16384, 512, 1000
PW = D // 32
NCHUNK_F32 = D // L
B_PER_W = BATCH // NW
CHUNK = 32
NG = B_PER_W // CHUNK
RPT = 64

_mesh = plsc.VectorSubcoreMesh(core_axis_name="c", subcore_axis_name="s")
_params = pltpu.CompilerParams(needs_layout_passes=False)

_BIT = [1 << c for c in range(32)]


@functools.partial(
    pl.kernel,
    out_type=jax.ShapeDtypeStruct((BATCH, D), jnp.float32),
    mesh=_mesh,
    compiler_params=_params,
    scratch_types=[
        pltpu.HBM((NC * V * PW,), jnp.int32),
        pltpu.VMEM((RPT, D), jnp.float32),
        pltpu.VMEM((RPT * PW,), jnp.int32),
        pltpu.VMEM((V * PW,), jnp.int32),
        pltpu.VMEM((B_PER_W,), jnp.int32),
        pltpu.VMEM((2, CHUNK, D), jnp.float32),
        pltpu.VMEM((2, CHUNK, D), jnp.float32),
        pltpu.SemaphoreType.DMA,
        pltpu.SemaphoreType.DMA,
        pltpu.SemaphoreType.DMA,
        pltpu.SemaphoreType.DMA,
        pltpu.SemaphoreType.DMA,
    ],
)
def _sc_kernel(x_hbm, lbl_hbm, w_hbm, out_hbm,
               packed_hbm, w_v, pk_v, ptab_v, lbl_v, x_v, out_v,
               sx0, sx1, so0, so1, sl):
    cid = lax.axis_index("c")
    sid = lax.axis_index("s")
    base_w = (sid * NC + cid) * B_PER_W
    lane_ids = lax.iota(jnp.int32, L)
    sx, so = (sx0, sx1), (so0, so1)

    def x_copy(g):
        return pltpu.make_async_copy(
            x_hbm.at[pl.ds(base_w + g * CHUNK, CHUNK)], x_v.at[g % 2],
            sx[g % 2])

    def out_copy(g):
        return pltpu.make_async_copy(
            out_v.at[g % 2], out_hbm.at[pl.ds(base_w + g * CHUNK, CHUNK)],
            so[g % 2])

    lbl_cp = pltpu.make_async_copy(
        lbl_hbm.at[pl.ds(base_w, B_PER_W)], lbl_v, sl)
    lbl_cp.start()
    x_copy(0).start()

    start = jnp.minimum(sid * RPT, V - RPT)
    pltpu.sync_copy(w_hbm.at[pl.ds(start, RPT)], w_v)

    def pack_row(r, _):
        bits = jnp.zeros((L,), jnp.uint32)
        for c in range(32):
            wv = w_v[r, pl.ds(c * L, L)]
            bits = bits | jnp.where(wv > 0.5, jnp.uint32(_BIT[c]),
                                    jnp.uint32(0))
        pk_v[pl.ds(r * PW, PW)] = plsc.bitcast(bits, jnp.int32)
        return 0

    lax.fori_loop(0, RPT, pack_row, 0)
    pltpu.sync_copy(pk_v,
                    packed_hbm.at[pl.ds(cid * V * PW + start * PW, RPT * PW)])
    plsc.subcore_barrier()
    pltpu.sync_copy(packed_hbm.at[pl.ds(cid * V * PW, V * PW)], ptab_v)
    lbl_cp.wait()

    for g in range(NG):
        b = g % 2
        if g + 1 < NG:
            x_copy(g + 1).start()
        x_copy(g).wait()
        if g >= 2:
            out_copy(g - 2).wait()

        def row_body(i, _):
            row_splat = plsc.load_gather(
                lbl_v, [jnp.full((L,), g * CHUNK + i, jnp.int32)])
            pv = plsc.bitcast(
                plsc.load_gather(ptab_v, [row_splat * PW + lane_ids]),
                jnp.uint32)
            for c in range(NCHUNK_F32):
                xv = x_v[b, i, pl.ds(c * L, L)]
                m = (pv & jnp.uint32(_BIT[c])) != 0
                out_v[b, i, pl.ds(c * L, L)] = jnp.where(m, xv, 0.0)
            return 0

        lax.fori_loop(0, CHUNK, row_body, 0)
        out_copy(g).start()
    out_copy(NG - 2).wait()
    out_copy(NG - 1).wait()


def kernel(x, label, weight):
    return _sc_kernel(x, label, weight)
